# baseline (speedup 1.0000x reference)
import functools

import jax
import jax.numpy as jnp
from jax import lax
from jax.experimental import pallas as pl
from jax.experimental.pallas import tpu as pltpu
from jax.experimental.pallas import tpu_sc as plsc

VOCAB = 100000
EMB = 768
BS = 4
SEQ = 2048
B = BS * SEQ

NC, NS = 2, 16
NW = NC * NS
BPW = B // NW
CH = 32
NBUF = 4
NCH = BPW // CH
LANES = 16
EMBV = EMB // LANES
PROWS = SEQ // NW
PCH = 32

_sc_mesh = plsc.VectorSubcoreMesh(
    core_axis_name="c", subcore_axis_name="s", num_cores=NC, num_subcores=NS
)


@functools.partial(
    pl.kernel,
    out_type=(
        jax.ShapeDtypeStruct((B, EMB), jnp.float32),
        jax.ShapeDtypeStruct((B,), jnp.int32),
    ),
    mesh=_sc_mesh,
    scratch_types=[
        [pltpu.VMEM((CH,), jnp.int32) for _ in range(NCH)],
        pltpu.VMEM((BPW,), jnp.int32),
        pltpu.VMEM((EMB,), jnp.float32),
        [pltpu.VMEM((CH, EMB), jnp.float32) for _ in range(NBUF)],
        [pltpu.SemaphoreType.DMA for _ in range(NBUF)],
        [pltpu.SemaphoreType.DMA for _ in range(NBUF)],
    ],
)
def _sc_embed(tok_hbm, table_hbm, mod_hbm, x_hbm, mask_hbm,
              idx_v, mask_v, mod_v, bufs, in_sems, out_sems):
    wid = lax.axis_index("s") * NC + lax.axis_index("c")
    base = wid * BPW

    for c in range(NCH):
        pltpu.sync_copy(tok_hbm.at[wid * NCH + c], idx_v[c])
    pltpu.sync_copy(mod_hbm, mod_v)

    def gather(c):
        return pltpu.async_copy(
            table_hbm.at[idx_v[c]],
            bufs[c % NBUF], in_sems[c % NBUF],
        )

    ins = [None] * NCH
    outs = [None] * NCH
    for c in range(NBUF - 1):
        ins[c] = gather(c)

    for c in range(NCH):
        for j in range(CH // LANES):
            t = idx_v[c][pl.ds(j * LANES, LANES)]
            mask_v[pl.ds(c * CH + j * LANES, LANES)] = (
                jnp.where(t > 0, 1, 0).astype(jnp.int32))
    pltpu.sync_copy(mask_v, mask_hbm.at[pl.ds(base, BPW)])

    mods = [mod_v[pl.ds(j * LANES, LANES)] for j in range(EMBV)]

    for c in range(NCH):
        if c >= 1:
            outs[c - 1].wait()
        if c + NBUF - 1 < NCH:
            ins[c + NBUF - 1] = gather(c + NBUF - 1)
        ins[c].wait()
        buf = bufs[c % NBUF]

        def add_mod(r, carry, buf=buf):
            for j in range(EMBV):
                plsc.addupdate(buf.at[r, pl.ds(j * LANES, LANES)], mods[j])
            return carry

        lax.fori_loop(0, CH, add_mod, 0)
        outs[c] = pltpu.async_copy(
            buf, x_hbm.at[pl.ds(base + c * CH, CH)], out_sems[c % NBUF]
        )
    outs[NCH - 1].wait()


def kernel(tokens, shared_embed, pos_emb_cache, modality_embedding):
    tok_flat = tokens.reshape(B // CH, CH)
    x_flat, mask_flat = _sc_embed(tok_flat, shared_embed, modality_embedding)
    x = x_flat.reshape(BS, SEQ, EMB)
    mask = mask_flat.reshape(BS, SEQ)
    pos_emb = jnp.broadcast_to(pos_emb_cache[None], (BS, SEQ, EMB))
    return (x, mask, pos_emb)


# speedup vs baseline: 1.2360x; 1.2360x over previous
import functools

import jax
import jax.numpy as jnp
from jax import lax
from jax.experimental import pallas as pl
from jax.experimental.pallas import tpu as pltpu
from jax.experimental.pallas import tpu_sc as plsc

VOCAB = 100000
EMB = 768
BS = 4
SEQ = 2048
B = BS * SEQ

NC, NS = 2, 16
NW = NC * NS
BPW = B // NW
CH = 32
NBUF = 5
NCH = BPW // CH
LANES = 16
EMBV = EMB // LANES
PROWS = SEQ // NW
PCH = 32

_sc_mesh = plsc.VectorSubcoreMesh(
    core_axis_name="c", subcore_axis_name="s", num_cores=NC, num_subcores=NS
)


@functools.partial(
    pl.kernel,
    out_type=(
        jax.ShapeDtypeStruct((B, EMB), jnp.float32),
        jax.ShapeDtypeStruct((B,), jnp.int32),
    ),
    mesh=_sc_mesh,
    scratch_types=[
        [pltpu.VMEM((CH,), jnp.int32) for _ in range(NCH)],
        pltpu.VMEM((BPW,), jnp.int32),
        pltpu.VMEM((EMB,), jnp.float32),
        [pltpu.VMEM((CH, EMB), jnp.float32) for _ in range(NBUF)],
        [pltpu.SemaphoreType.DMA for _ in range(NBUF)],
        [pltpu.SemaphoreType.DMA for _ in range(NBUF)],
    ],
)
def _sc_embed(tok_hbm, table_hbm, mod_hbm, x_hbm, mask_hbm,
              idx_v, mask_v, mod_v, bufs, in_sems, out_sems):
    wid = lax.axis_index("s") * NC + lax.axis_index("c")
    base = wid * BPW

    for c in range(NCH):
        pltpu.sync_copy(tok_hbm.at[wid * NCH + c], idx_v[c])
    pltpu.sync_copy(mod_hbm, mod_v)

    def gather(c):
        return pltpu.async_copy(
            table_hbm.at[idx_v[c]],
            bufs[c % NBUF], in_sems[c % NBUF],
        )

    ins = [None] * NCH
    outs = [None] * NCH
    for c in range(NBUF - 1):
        ins[c] = gather(c)

    for c in range(NCH):
        for j in range(CH // LANES):
            t = idx_v[c][pl.ds(j * LANES, LANES)]
            mask_v[pl.ds(c * CH + j * LANES, LANES)] = (
                jnp.where(t > 0, 1, 0).astype(jnp.int32))
    pltpu.sync_copy(mask_v, mask_hbm.at[pl.ds(base, BPW)])

    mods = [mod_v[pl.ds(j * LANES, LANES)] for j in range(EMBV)]

    for c in range(NCH):
        ins[c].wait()
        buf = bufs[c % NBUF]

        def add_mod(r, carry, buf=buf):
            for j in range(EMBV):
                plsc.addupdate(buf.at[r, pl.ds(j * LANES, LANES)], mods[j])
            return carry

        lax.fori_loop(0, CH, add_mod, 0)
        outs[c] = pltpu.async_copy(
            buf, x_hbm.at[pl.ds(base + c * CH, CH)], out_sems[c % NBUF]
        )
        g = c + NBUF - 1
        if g < NCH:
            if g >= NBUF:
                outs[g - NBUF].wait()
            ins[g] = gather(g)
    for c in range(max(0, NCH - NBUF), NCH):
        outs[c].wait()


def _pos_body(cache_ref, out_ref):
    out_ref[...] = jnp.broadcast_to(cache_ref[...][None], out_ref.shape)


_POS_BLK = 512


def _pos_broadcast(pos_emb_cache):
    return pl.pallas_call(
        _pos_body,
        grid=(SEQ // _POS_BLK,),
        in_specs=[pl.BlockSpec((_POS_BLK, EMB), lambda i: (i, 0))],
        out_specs=pl.BlockSpec((BS, _POS_BLK, EMB), lambda i: (0, i, 0)),
        out_shape=jax.ShapeDtypeStruct((BS, SEQ, EMB), jnp.float32),
    )(pos_emb_cache)


def kernel(tokens, shared_embed, pos_emb_cache, modality_embedding):
    tok_flat = tokens.reshape(B // CH, CH)
    x_flat, mask_flat = _sc_embed(tok_flat, shared_embed, modality_embedding)
    x = x_flat.reshape(BS, SEQ, EMB)
    mask = mask_flat.reshape(BS, SEQ)
    pos_emb = _pos_broadcast(pos_emb_cache)
    return (x, mask, pos_emb)


# speedup vs baseline: 1.2486x; 1.0102x over previous
import functools

import jax
import jax.numpy as jnp
from jax import lax
from jax.experimental import pallas as pl
from jax.experimental.pallas import tpu as pltpu
from jax.experimental.pallas import tpu_sc as plsc

VOCAB = 100000
EMB = 768
BS = 4
SEQ = 2048
B = BS * SEQ

NC, NS = 2, 16
NW = NC * NS
BPW = B // NW
CH = 32
NBUF = 5
NCH = BPW // CH
LANES = 16
EMBV = EMB // LANES
PROWS = SEQ // NW
PCH = 32

_sc_mesh = plsc.VectorSubcoreMesh(
    core_axis_name="c", subcore_axis_name="s", num_cores=NC, num_subcores=NS
)


@functools.partial(
    pl.kernel,
    out_type=(
        jax.ShapeDtypeStruct((B, EMB), jnp.float32),
        jax.ShapeDtypeStruct((B,), jnp.int32),
    ),
    mesh=_sc_mesh,
    scratch_types=[
        [pltpu.VMEM((CH,), jnp.int32) for _ in range(NCH)],
        pltpu.VMEM((BPW,), jnp.int32),
        pltpu.VMEM((EMB,), jnp.float32),
        [pltpu.VMEM((CH, EMB), jnp.float32) for _ in range(NBUF)],
        [pltpu.SemaphoreType.DMA for _ in range(NBUF)],
        [pltpu.SemaphoreType.DMA for _ in range(NBUF)],
    ],
)
def _sc_embed(tok_hbm, table_hbm, mod_hbm, x_hbm, mask_hbm,
              idx_v, mask_v, mod_v, bufs, in_sems, out_sems):
    wid = lax.axis_index("s") * NC + lax.axis_index("c")
    base = wid * BPW

    for c in range(NCH):
        pltpu.sync_copy(tok_hbm.at[wid * NCH + c], idx_v[c])
    pltpu.sync_copy(mod_hbm, mod_v)

    def gather(c):
        return pltpu.async_copy(
            table_hbm.at[idx_v[c]],
            bufs[c % NBUF], in_sems[c % NBUF],
        )

    ins = [None] * NCH
    outs = [None] * NCH
    for c in range(NBUF - 1):
        ins[c] = gather(c)

    for c in range(NCH):
        for j in range(CH // LANES):
            t = idx_v[c][pl.ds(j * LANES, LANES)]
            mask_v[pl.ds(c * CH + j * LANES, LANES)] = (
                jnp.where(t > 0, 1, 0).astype(jnp.int32))
    pltpu.sync_copy(mask_v, mask_hbm.at[pl.ds(base, BPW)])

    mods = [mod_v[pl.ds(j * LANES, LANES)] for j in range(EMBV)]

    for c in range(NCH):
        ins[c].wait()
        buf = bufs[c % NBUF]

        def add_mod(r, carry, buf=buf):
            for j in range(EMBV):
                plsc.addupdate(buf.at[r, pl.ds(j * LANES, LANES)], mods[j])
            return carry

        lax.fori_loop(0, CH, add_mod, 0)
        outs[c] = pltpu.async_copy(
            buf, x_hbm.at[pl.ds(base + c * CH, CH)], out_sems[c % NBUF]
        )
        g = c + NBUF - 1
        if g < NCH:
            if g >= NBUF:
                outs[g - NBUF].wait()
            ins[g] = gather(g)
    for c in range(max(0, NCH - NBUF), NCH):
        outs[c].wait()


def _pos_body(cache_ref, out_ref):
    out_ref[...] = jnp.broadcast_to(cache_ref[...][None], out_ref.shape)


_POS_BLK = 1024


def _pos_broadcast(pos_emb_cache):
    return pl.pallas_call(
        _pos_body,
        grid=(SEQ // _POS_BLK,),
        in_specs=[pl.BlockSpec((_POS_BLK, EMB), lambda i: (i, 0))],
        out_specs=pl.BlockSpec((BS, _POS_BLK, EMB), lambda i: (0, i, 0)),
        out_shape=jax.ShapeDtypeStruct((BS, SEQ, EMB), jnp.float32),
    )(pos_emb_cache)


def kernel(tokens, shared_embed, pos_emb_cache, modality_embedding):
    tok_flat = tokens.reshape(B // CH, CH)
    x_flat, mask_flat = _sc_embed(tok_flat, shared_embed, modality_embedding)
    x = x_flat.reshape(BS, SEQ, EMB)
    mask = mask_flat.reshape(BS, SEQ)
    pos_emb = _pos_broadcast(pos_emb_cache)
    return (x, mask, pos_emb)
